# baseline (speedup 1.0000x reference)
import functools

import jax
import jax.numpy as jnp
from jax import lax
from jax.experimental import pallas as pl
from jax.experimental.pallas import tpu as pltpu
from jax.experimental.pallas import tpu_sc as plsc

L = 16
KK = 128
KV = KK // L
ROW = 8192
NROWS = 128 * 32
GROUPS = ROW // KK
NC = 2
NS = 16
NW = NC * NS
RPW = NROWS // NW


def _vsort(v, desc):
    k, _ = plsc.sort_key_val(v, v, descending=desc)
    return k


def _finish(C, desc):
    n = len(C)
    d = n // 2
    while d >= 1:
        for s in range(0, n, 2 * d):
            for i in range(s, s + d):
                a, b = C[i], C[i + d]
                hi, lo = jnp.maximum(a, b), jnp.minimum(a, b)
                C[i], C[i + d] = (hi, lo) if desc else (lo, hi)
        d //= 2
    return [_vsort(c, desc) for c in C]


def _merge_full(A, B, desc):
    return _finish(list(A) + list(B), desc)


def _merge_capped(A, B, desc):
    H = [jnp.maximum(a, b) for a, b in zip(A, B)]
    return _finish(H, desc)


def _sc_topk(x_hbm, out_hbm, row_a, row_b, wa, wb, outs, sem_a, sem_b):
    wid = lax.axis_index("s") * NC + lax.axis_index("c")
    base = wid * RPW

    def build_run(row_v, goff, flip):
        runs = [[_vsort(row_v[pl.ds(goff + k * L, L)],
                        desc=((k % 2 == 0) != flip))] for k in range(KV)]
        while len(runs) > 1:
            runs = [_merge_full(runs[2 * t], runs[2 * t + 1],
                                desc=((t % 2 == 0) != flip))
                    for t in range(len(runs) // 2)]
        return runs[0]

    def process_row(row_v, j):
        @plsc.parallel_loop(0, GROUPS // 2, 1, unroll=2)
        def group_body(u):
            for parity in range(2):
                g = 2 * u + parity
                R = build_run(row_v, g * KK, flip=(parity == 1))
                for k in range(KV):
                    wa[pl.ds(g * KK + k * L, L)] = R[k]

        cur, nxt = wa, wb
        for rnd in range(5):
            n_out = GROUPS >> (rnd + 1)

            def _make_cap_body(cur, nxt):
                def cap_body(u):
                    for parity in range(2):
                        i = 2 * u + parity
                        A = [cur[pl.ds((2 * i) * KK + k * L, L)]
                             for k in range(KV)]
                        B = [cur[pl.ds((2 * i + 1) * KK + k * L, L)]
                             for k in range(KV)]
                        R = _merge_capped(A, B, desc=(parity == 0))
                        for k in range(KV):
                            nxt[pl.ds(i * KK + k * L, L)] = R[k]
                return cap_body

            plsc.parallel_loop(0, n_out // 2, 1, unroll=min(2, n_out // 2))(
                _make_cap_body(cur, nxt))
            cur, nxt = nxt, cur

        A = [cur[pl.ds(k * L, L)] for k in range(KV)]
        B = [cur[pl.ds(KK + k * L, L)] for k in range(KV)]
        R = _merge_capped(A, B, desc=True)
        for k in range(KV):
            outs[j, pl.ds(k * L, L)] = R[k]

    pltpu.make_async_copy(x_hbm.at[base], row_a, sem_a).start()

    def pair_body(p, carry):
        r0 = base + 2 * p
        pltpu.make_async_copy(x_hbm.at[r0 + 1], row_b, sem_b).start()
        pltpu.make_async_copy(x_hbm.at[r0], row_a, sem_a).wait()
        process_row(row_a, 2 * p)
        nxt = base + ((2 * p + 2) & (RPW - 1))
        pltpu.make_async_copy(x_hbm.at[nxt], row_a, sem_a).start()
        pltpu.make_async_copy(x_hbm.at[r0 + 1], row_b, sem_b).wait()
        process_row(row_b, 2 * p + 1)
        return carry

    lax.fori_loop(0, RPW // 2, pair_body, 0)
    pltpu.make_async_copy(x_hbm.at[base], row_a, sem_a).wait()
    pltpu.sync_copy(outs, out_hbm.at[pl.ds(base, RPW)])


_mesh = plsc.VectorSubcoreMesh(
    core_axis_name="c", subcore_axis_name="s", num_cores=NC, num_subcores=NS)

_topk_call = functools.partial(
    pl.kernel,
    out_type=jax.ShapeDtypeStruct((NROWS, KK), jnp.float32),
    mesh=_mesh,
    compiler_params=pltpu.CompilerParams(needs_layout_passes=False),
    scratch_types=[
        pltpu.VMEM((ROW,), jnp.float32),
        pltpu.VMEM((ROW,), jnp.float32),
        pltpu.VMEM((ROW,), jnp.float32),
        pltpu.VMEM((ROW,), jnp.float32),
        pltpu.VMEM((RPW, KK), jnp.float32),
        pltpu.SemaphoreType.DMA,
        pltpu.SemaphoreType.DMA,
    ],
)(_sc_topk)


@jax.jit
def kernel(input):
    x = input.reshape(NROWS, ROW)
    out = _topk_call(x)
    return out.reshape(128, 32, KK)


# speedup vs baseline: 1.1073x; 1.1073x over previous
import functools

import jax
import jax.numpy as jnp
from jax import lax
from jax.experimental import pallas as pl
from jax.experimental.pallas import tpu as pltpu
from jax.experimental.pallas import tpu_sc as plsc

L = 16
KK = 128
KV = KK // L
ROW = 8192
NROWS = 128 * 32
GROUPS = ROW // KK
NC = 2
NS = 16
NW = NC * NS
RPW = NROWS // NW


def _vsort(v, desc):
    k, _ = plsc.sort_key_val(v, v, descending=desc)
    return k


def _finish(C, desc):
    n = len(C)
    d = n // 2
    while d >= 1:
        for s in range(0, n, 2 * d):
            for i in range(s, s + d):
                a, b = C[i], C[i + d]
                hi, lo = jnp.maximum(a, b), jnp.minimum(a, b)
                C[i], C[i + d] = (hi, lo) if desc else (lo, hi)
        d //= 2
    return [_vsort(c, desc) for c in C]


def _merge_full(A, B, desc):
    return _finish(list(A) + list(B), desc)


def _merge_capped(A, B, desc):
    H = [jnp.maximum(a, b) for a, b in zip(A, B)]
    return _finish(H, desc)


def _sc_topk(x_hbm, out_hbm, row_a, row_b, wa, wb, outs, sem_a, sem_b):
    wid = lax.axis_index("s") * NC + lax.axis_index("c")
    base = wid * RPW

    def build_run(row_v, goff, flip):
        runs = [[_vsort(row_v[pl.ds(goff + k * L, L)],
                        desc=((k % 2 == 0) != flip))] for k in range(KV)]
        while len(runs) > 1:
            runs = [_merge_full(runs[2 * t], runs[2 * t + 1],
                                desc=((t % 2 == 0) != flip))
                    for t in range(len(runs) // 2)]
        return runs[0]

    def process_row(row_v, j):
        for parity in range(2):
            def _make_group_body(parity):
                def group_body(u):
                    g = 2 * u + parity
                    R = build_run(row_v, g * KK, flip=(parity == 1))
                    for k in range(KV):
                        wa[pl.ds(g * KK + k * L, L)] = R[k]
                return group_body

            plsc.parallel_loop(0, GROUPS // 2, 1, unroll=2)(
                _make_group_body(parity))

        cur, nxt = wa, wb
        for rnd in range(5):
            n_out = GROUPS >> (rnd + 1)

            def _make_cap_body(cur, nxt, parity):
                def cap_body(u):
                    i = 2 * u + parity
                    A = [cur[pl.ds((2 * i) * KK + k * L, L)]
                         for k in range(KV)]
                    B = [cur[pl.ds((2 * i + 1) * KK + k * L, L)]
                         for k in range(KV)]
                    R = _merge_capped(A, B, desc=(parity == 0))
                    for k in range(KV):
                        nxt[pl.ds(i * KK + k * L, L)] = R[k]
                return cap_body

            for parity in range(2):
                plsc.parallel_loop(0, n_out // 2, 1,
                                   unroll=min(2, n_out // 2))(
                    _make_cap_body(cur, nxt, parity))
            cur, nxt = nxt, cur

        A = [cur[pl.ds(k * L, L)] for k in range(KV)]
        B = [cur[pl.ds(KK + k * L, L)] for k in range(KV)]
        R = _merge_capped(A, B, desc=True)
        for k in range(KV):
            outs[j, pl.ds(k * L, L)] = R[k]

    pltpu.make_async_copy(x_hbm.at[base], row_a, sem_a).start()

    def pair_body(p, carry):
        r0 = base + 2 * p
        pltpu.make_async_copy(x_hbm.at[r0 + 1], row_b, sem_b).start()
        pltpu.make_async_copy(x_hbm.at[r0], row_a, sem_a).wait()
        process_row(row_a, 2 * p)
        nxt = base + ((2 * p + 2) & (RPW - 1))
        pltpu.make_async_copy(x_hbm.at[nxt], row_a, sem_a).start()
        pltpu.make_async_copy(x_hbm.at[r0 + 1], row_b, sem_b).wait()
        process_row(row_b, 2 * p + 1)
        return carry

    lax.fori_loop(0, RPW // 2, pair_body, 0)
    pltpu.make_async_copy(x_hbm.at[base], row_a, sem_a).wait()
    pltpu.sync_copy(outs, out_hbm.at[pl.ds(base, RPW)])


_mesh = plsc.VectorSubcoreMesh(
    core_axis_name="c", subcore_axis_name="s", num_cores=NC, num_subcores=NS)

_topk_call = functools.partial(
    pl.kernel,
    out_type=jax.ShapeDtypeStruct((NROWS, KK), jnp.float32),
    mesh=_mesh,
    compiler_params=pltpu.CompilerParams(needs_layout_passes=False),
    scratch_types=[
        pltpu.VMEM((ROW,), jnp.float32),
        pltpu.VMEM((ROW,), jnp.float32),
        pltpu.VMEM((ROW,), jnp.float32),
        pltpu.VMEM((ROW,), jnp.float32),
        pltpu.VMEM((RPW, KK), jnp.float32),
        pltpu.SemaphoreType.DMA,
        pltpu.SemaphoreType.DMA,
    ],
)(_sc_topk)


@jax.jit
def kernel(input):
    x = input.reshape(NROWS, ROW)
    out = _topk_call(x)
    return out.reshape(128, 32, KK)


# speedup vs baseline: 1.3493x; 1.2185x over previous
import functools

import jax
import jax.numpy as jnp
from jax import lax
from jax.experimental import pallas as pl
from jax.experimental.pallas import tpu as pltpu
from jax.experimental.pallas import tpu_sc as plsc

L = 16
KK = 128
KV = KK // L
ROW = 8192
NROWS = 128 * 32
GROUPS = ROW // KK
NC = 2
NS = 16
NW = NC * NS
RPW = NROWS // NW


def _vsort(v, desc):
    k, _ = plsc.sort_key_val(v, v, descending=desc)
    return k


def _finish(C, desc):
    n = len(C)
    d = n // 2
    while d >= 1:
        for s in range(0, n, 2 * d):
            for i in range(s, s + d):
                a, b = C[i], C[i + d]
                hi, lo = jnp.maximum(a, b), jnp.minimum(a, b)
                C[i], C[i + d] = (hi, lo) if desc else (lo, hi)
        d //= 2
    return [_vsort(c, desc) for c in C]


def _build_run_desc(Y):
    runs = [[_vsort(Y[k], desc=(k % 2 == 0))] for k in range(KV)]
    while len(runs) > 1:
        runs = [_finish(runs[2 * t] + runs[2 * t + 1], desc=(t % 2 == 0))
                for t in range(len(runs) // 2)]
    return runs[0]


def _parity_sign(i):
    return 1.0 - 2.0 * lax.convert_element_type(
        jnp.bitwise_and(i, 1), jnp.float32)


def _sc_topk(x_hbm, out_hbm, row_a, row_b, wa, wb, outs, sem_a, sem_b):
    wid = lax.axis_index("s") * NC + lax.axis_index("c")
    base = wid * RPW

    def process_row(row_v, j):
        @plsc.parallel_loop(0, GROUPS, 1, unroll=2)
        def group_body(g):
            sf = _parity_sign(g)
            Y = [row_v[pl.ds(g * KK + k * L, L)] * sf for k in range(KV)]
            R = _build_run_desc(Y)
            for k in range(KV):
                wa[pl.ds(g * KK + k * L, L)] = R[k] * sf

        cur, nxt = wa, wb
        for rnd in range(5):
            n_out = GROUPS >> (rnd + 1)

            def _make_cap_body(cur, nxt):
                def cap_body(i):
                    sf = _parity_sign(i)
                    H = [jnp.maximum(cur[pl.ds((2 * i) * KK + k * L, L)],
                                     cur[pl.ds((2 * i + 1) * KK + k * L, L)])
                         * sf for k in range(KV)]
                    R = _finish(H, desc=True)
                    for k in range(KV):
                        nxt[pl.ds(i * KK + k * L, L)] = R[k] * sf
                return cap_body

            plsc.parallel_loop(0, n_out, 1, unroll=min(2, n_out))(
                _make_cap_body(cur, nxt))
            cur, nxt = nxt, cur

        H = [jnp.maximum(cur[pl.ds(k * L, L)], cur[pl.ds(KK + k * L, L)])
             for k in range(KV)]
        R = _finish(H, desc=True)
        for k in range(KV):
            outs[j, pl.ds(k * L, L)] = R[k]

    pltpu.make_async_copy(x_hbm.at[base], row_a, sem_a).start()

    def pair_body(p, carry):
        r0 = base + 2 * p
        pltpu.make_async_copy(x_hbm.at[r0 + 1], row_b, sem_b).start()
        pltpu.make_async_copy(x_hbm.at[r0], row_a, sem_a).wait()
        process_row(row_a, 2 * p)
        nxt = base + ((2 * p + 2) & (RPW - 1))
        pltpu.make_async_copy(x_hbm.at[nxt], row_a, sem_a).start()
        pltpu.make_async_copy(x_hbm.at[r0 + 1], row_b, sem_b).wait()
        process_row(row_b, 2 * p + 1)
        return carry

    lax.fori_loop(0, RPW // 2, pair_body, 0)
    pltpu.make_async_copy(x_hbm.at[base], row_a, sem_a).wait()
    pltpu.sync_copy(outs, out_hbm.at[pl.ds(base, RPW)])


_mesh = plsc.VectorSubcoreMesh(
    core_axis_name="c", subcore_axis_name="s", num_cores=NC, num_subcores=NS)

_topk_call = functools.partial(
    pl.kernel,
    out_type=jax.ShapeDtypeStruct((NROWS, KK), jnp.float32),
    mesh=_mesh,
    compiler_params=pltpu.CompilerParams(needs_layout_passes=False),
    scratch_types=[
        pltpu.VMEM((ROW,), jnp.float32),
        pltpu.VMEM((ROW,), jnp.float32),
        pltpu.VMEM((ROW,), jnp.float32),
        pltpu.VMEM((ROW,), jnp.float32),
        pltpu.VMEM((RPW, KK), jnp.float32),
        pltpu.SemaphoreType.DMA,
        pltpu.SemaphoreType.DMA,
    ],
)(_sc_topk)


@jax.jit
def kernel(input):
    x = input.reshape(NROWS, ROW)
    out = _topk_call(x)
    return out.reshape(128, 32, KK)


# speedup vs baseline: 1.3729x; 1.0175x over previous
import functools

import jax
import jax.numpy as jnp
from jax import lax
from jax.experimental import pallas as pl
from jax.experimental.pallas import tpu as pltpu
from jax.experimental.pallas import tpu_sc as plsc

L = 16
KK = 128
KV = KK // L
ROW = 8192
NROWS = 128 * 32
GROUPS = ROW // KK
NC = 2
NS = 16
NW = NC * NS
RPW = NROWS // NW


def _vsort(v, desc):
    k, _ = plsc.sort_key_val(v, v, descending=desc)
    return k


def _finish(C, desc):
    n = len(C)
    d = n // 2
    while d >= 1:
        for s in range(0, n, 2 * d):
            for i in range(s, s + d):
                a, b = C[i], C[i + d]
                hi, lo = jnp.maximum(a, b), jnp.minimum(a, b)
                C[i], C[i + d] = (hi, lo) if desc else (lo, hi)
        d //= 2
    return [_vsort(c, desc) for c in C]


def _build_run_desc(Y):
    runs = [[_vsort(Y[k], desc=(k % 2 == 0))] for k in range(KV)]
    while len(runs) > 1:
        runs = [_finish(runs[2 * t] + runs[2 * t + 1], desc=(t % 2 == 0))
                for t in range(len(runs) // 2)]
    return runs[0]


def _parity_sign(i):
    return 1.0 - 2.0 * lax.convert_element_type(
        jnp.bitwise_and(i, 1), jnp.float32)


def _sc_topk(x_hbm, out_hbm, row_a, row_b, wa, wb, outs, sem_a, sem_b):
    wid = lax.axis_index("s") * NC + lax.axis_index("c")
    base = wid * RPW

    def process_row(row_v, j):
        @plsc.parallel_loop(0, GROUPS, 1, unroll=2)
        def group_body(g):
            sf = _parity_sign(g)
            Y = [row_v[pl.ds(g * KK + k * L, L)] * sf for k in range(KV)]
            R = _build_run_desc(Y)
            for k in range(KV):
                wa[pl.ds(g * KK + k * L, L)] = R[k] * sf

        cur, nxt = wa, wb
        for rnd in range(3):
            n_out = GROUPS >> (rnd + 1)

            def _make_cap_body(cur, nxt):
                def cap_body(i):
                    sf = _parity_sign(i)
                    H = [jnp.maximum(cur[pl.ds((2 * i) * KK + k * L, L)],
                                     cur[pl.ds((2 * i + 1) * KK + k * L, L)])
                         * sf for k in range(KV)]
                    R = _finish(H, desc=True)
                    for k in range(KV):
                        nxt[pl.ds(i * KK + k * L, L)] = R[k] * sf
                return cap_body

            plsc.parallel_loop(0, n_out, 1, unroll=min(2, n_out))(
                _make_cap_body(cur, nxt))
            cur, nxt = nxt, cur

        for n_out in (4, 2):
            for i in range(n_out):
                H = [jnp.maximum(cur[pl.ds((2 * i) * KK + k * L, L)],
                                 cur[pl.ds((2 * i + 1) * KK + k * L, L)])
                     for k in range(KV)]
                R = _finish(H, desc=(i % 2 == 0))
                for k in range(KV):
                    nxt[pl.ds(i * KK + k * L, L)] = R[k]
            cur, nxt = nxt, cur

        H = [jnp.maximum(cur[pl.ds(k * L, L)], cur[pl.ds(KK + k * L, L)])
             for k in range(KV)]
        R = _finish(H, desc=True)
        for k in range(KV):
            outs[j, pl.ds(k * L, L)] = R[k]

    pltpu.make_async_copy(x_hbm.at[base], row_a, sem_a).start()

    def pair_body(p, carry):
        r0 = base + 2 * p
        pltpu.make_async_copy(x_hbm.at[r0 + 1], row_b, sem_b).start()
        pltpu.make_async_copy(x_hbm.at[r0], row_a, sem_a).wait()
        process_row(row_a, 2 * p)
        nxt = base + ((2 * p + 2) & (RPW - 1))
        pltpu.make_async_copy(x_hbm.at[nxt], row_a, sem_a).start()
        pltpu.make_async_copy(x_hbm.at[r0 + 1], row_b, sem_b).wait()
        process_row(row_b, 2 * p + 1)
        return carry

    lax.fori_loop(0, RPW // 2, pair_body, 0)
    pltpu.make_async_copy(x_hbm.at[base], row_a, sem_a).wait()
    pltpu.sync_copy(outs, out_hbm.at[pl.ds(base, RPW)])


_mesh = plsc.VectorSubcoreMesh(
    core_axis_name="c", subcore_axis_name="s", num_cores=NC, num_subcores=NS)

_topk_call = functools.partial(
    pl.kernel,
    out_type=jax.ShapeDtypeStruct((NROWS, KK), jnp.float32),
    mesh=_mesh,
    compiler_params=pltpu.CompilerParams(needs_layout_passes=False),
    scratch_types=[
        pltpu.VMEM((ROW,), jnp.float32),
        pltpu.VMEM((ROW,), jnp.float32),
        pltpu.VMEM((ROW,), jnp.float32),
        pltpu.VMEM((ROW,), jnp.float32),
        pltpu.VMEM((RPW, KK), jnp.float32),
        pltpu.SemaphoreType.DMA,
        pltpu.SemaphoreType.DMA,
    ],
)(_sc_topk)


@jax.jit
def kernel(input):
    x = input.reshape(NROWS, ROW)
    out = _topk_call(x)
    return out.reshape(128, 32, KK)
